# baseline (speedup 1.0000x reference)
import jax
import jax.numpy as jnp
from jax.experimental import pallas as pl
from jax.experimental.pallas import tpu as pltpu

_PACK = 16
_TILE_M = 2048


def _fused_body(x_ref, z_ref, wx_ref, wz_ref, b_ref, o_ref):
    o_ref[...] = (
        jnp.dot(x_ref[...], wx_ref[...], preferred_element_type=jnp.float32)
        + jnp.dot(z_ref[...], wz_ref[...], preferred_element_type=jnp.float32)
        + b_ref[...]
    )


def kernel(x, z, wxt, wzt, bt):
    B, depth = x.shape
    pack = _PACK
    lanes = pack * depth

    eye = jnp.eye(pack, dtype=jnp.float32)
    wx_big = jnp.kron(eye, wxt.T)
    wz_big = jnp.kron(eye, wzt.T)
    b_big = jnp.tile(bt.reshape(1, depth), (1, pack))

    M = B // pack
    x2 = x.reshape(M, lanes)
    z2 = z.reshape(M, lanes)

    grid = (pl.cdiv(M, _TILE_M),)
    row_spec = pl.BlockSpec((_TILE_M, lanes), lambda i: (i, 0))
    w_spec = pl.BlockSpec((lanes, lanes), lambda i: (0, 0))
    b_spec = pl.BlockSpec((1, lanes), lambda i: (0, 0))

    out2 = pl.pallas_call(
        _fused_body,
        out_shape=jax.ShapeDtypeStruct((M, lanes), jnp.float32),
        grid=grid,
        in_specs=[row_spec, row_spec, w_spec, w_spec, b_spec],
        out_specs=row_spec,
        compiler_params=pltpu.CompilerParams(dimension_semantics=("parallel",)),
    )(x2, z2, wx_big, wz_big, b_big)

    return out2.reshape(B, depth)
